# Initial kernel scaffold; baseline (speedup 1.0000x reference)
import jax
import jax.numpy as jnp
from jax.experimental import pallas as pl


def kernel(depth, extrinsics, intrinsics, tsdf_volume, feature_volume, origin, resolution, gpu, weights_volume):
    raise NotImplementedError("write your pallas kernel here")


# speedup vs baseline: 2.0588x; 2.0588x over previous
import functools

import jax
import jax.numpy as jnp
from jax import lax
from jax.experimental import pallas as pl
from jax.experimental.pallas import tpu as pltpu
from jax.experimental.pallas import tpu_sc as plsc

N_POINTS = 9
N_HALF = (N_POINTS - 1) // 2

H = 480
W = 640
RB = 16
NPIX = H * W
NPTS = NPIX * N_POINTS
VS = 256
NW = 32
PER_TILE = NPTS // NW
CHUNK = 1920
NCHUNKS = PER_TILE // CHUNK


def _geom_body(pp_ref, co_ref, idx_ref, wt_ref):
    P = lambda j: pp_ref[0, j]
    cox = co_ref[0]
    coy = co_ref[1]
    coz = co_ref[2]
    o0, o1, o2, res = P(3), P(4), P(5), P(6)
    cenx = (cox - o0) / res
    ceny = (coy - o1) / res
    cenz = (coz - o2) / res
    eyex = (P(0) - o0) / res
    eyey = (P(1) - o1) / res
    eyez = (P(2) - o2) / res
    dx = cenx - eyex
    dy = ceny - eyey
    dz = cenz - eyez
    nrm = jnp.sqrt(dx * dx + dy * dy + dz * dz)
    inv = 1.0 / jnp.maximum(nrm, 1e-12)
    dx = dx * inv
    dy = dy * inv
    dz = dz * inv

    def corner_data(p, size, scale):
        f = jnp.floor(p)
        a = p - f
        nb = jnp.where(a > 0, 1.0, 0.0)
        g0 = f
        g1 = f + nb
        out = []
        for g, wgt in ((g0, 1.0 - a), (g1, a)):
            valid = (g >= 0) & (g < size)
            gi = jnp.minimum(jnp.maximum(g, 0.0), size - 1).astype(jnp.int32)
            gi = jnp.clip(gi, 0, size - 1) * scale
            out.append((wgt, valid, gi))
        return out

    for n in range(N_POINTS):
        off = float(n - N_HALF)
        px = cenx + off * dx
        py = ceny + off * dy
        pz = cenz + off * dz
        X = corner_data(px, VS, VS * VS)
        Y = corner_data(py, VS, VS)
        Z = corner_data(pz, VS, 1)
        cidx = 0
        for ci in (0, 1):
            wx, vx, ix = X[ci]
            for cj in (0, 1):
                wy, vy, iy = Y[cj]
                wxy = wx * wy
                vxy = vx & vy
                ixy = ix + iy
                for ck in (0, 1):
                    wz, vz, iz = Z[ck]
                    wt = jnp.where(vxy & vz, wxy * wz, 0.0)
                    idx_ref[cidx, n] = ixy + iz
                    wt_ref[cidx, n] = wt
                    cidx += 1


def _geom_call(params, coords_t):
    shape = (8, N_POINTS, H, W)
    return pl.pallas_call(
        _geom_body,
        grid=(H // RB,),
        in_specs=[
            pl.BlockSpec((1, 32), lambda i: (0, 0)),
            pl.BlockSpec((3, RB, W), lambda i: (0, i, 0)),
        ],
        out_specs=[
            pl.BlockSpec((8, N_POINTS, RB, W), lambda i: (0, 0, i, 0)),
            pl.BlockSpec((8, N_POINTS, RB, W), lambda i: (0, 0, i, 0)),
        ],
        out_shape=[
            jax.ShapeDtypeStruct(shape, jnp.int32),
            jax.ShapeDtypeStruct(shape, jnp.float32),
        ],
    )(params, coords_t)


def _sc_body(idx_hbm, wt_hbm, ts_hbm, wv_hbm, out_hbm, *refs):
    idx_v = refs[0:8]
    wt_v = refs[8:16]
    gv = refs[16:24]
    gw = refs[24:32]
    accv, accw, insem, gsem = refs[32:36]
    wid = lax.axis_index("s") * 2 + lax.axis_index("c")
    base = wid * PER_TILE

    def chunk_body(ch, _):
        off = pl.multiple_of(base + ch * CHUNK, 128)
        stage = []
        for k in range(8):
            stage.append(pltpu.async_copy(
                idx_hbm.at[k, pl.ds(off, CHUNK)], idx_v[k], insem))
            stage.append(pltpu.async_copy(
                wt_hbm.at[k, pl.ds(off, CHUNK)], wt_v[k], insem))
        for cp in stage:
            cp.wait()
        copies = []
        for k in range(8):
            copies.append(pltpu.async_copy(ts_hbm.at[idx_v[k]], gv[k], gsem))
            copies.append(pltpu.async_copy(wv_hbm.at[idx_v[k]], gw[k], gsem))
        for cp in copies:
            cp.wait()

        def inner(i, _):
            s = pl.ds(i * 16, 16)
            av = jnp.zeros((16,), jnp.float32)
            aw = jnp.zeros((16,), jnp.float32)
            for k in range(8):
                wtk = wt_v[k][s]
                av = av + wtk * gv[k][s]
                aw = aw + wtk * gw[k][s]
            accv[s] = av
            accw[s] = aw
            return 0

        lax.fori_loop(0, CHUNK // 16, inner, 0)
        pltpu.sync_copy(accv, out_hbm.at[0, pl.ds(off, CHUNK)])
        pltpu.sync_copy(accw, out_hbm.at[1, pl.ds(off, CHUNK)])
        return 0

    lax.fori_loop(0, NCHUNKS, chunk_body, 0)


@functools.cache
def _sc_call():
    scratch = (
        [pltpu.VMEM((CHUNK,), jnp.int32) for _ in range(8)]
        + [pltpu.VMEM((CHUNK,), jnp.float32) for _ in range(24)]
        + [pltpu.VMEM((CHUNK,), jnp.float32) for _ in range(2)]
        + [pltpu.SemaphoreType.DMA, pltpu.SemaphoreType.DMA]
    )
    return functools.partial(
        pl.kernel,
        mesh=plsc.VectorSubcoreMesh(core_axis_name="c", subcore_axis_name="s"),
        out_type=jax.ShapeDtypeStruct((2, NPTS), jnp.float32),
        scratch_types=scratch,
    )(_sc_body)


def _camera_coords(depth, extrinsics, intrinsics):
    b, h, w = depth.shape
    xx, yy = jnp.meshgrid(jnp.arange(h, dtype=jnp.float32),
                          jnp.arange(w, dtype=jnp.float32), indexing='ij')
    xx = jnp.broadcast_to(xx.reshape(1, h * w, 1), (b, h * w, 1))
    yy = jnp.broadcast_to(yy.reshape(1, h * w, 1), (b, h * w, 1))
    zz = depth.reshape(b, h * w, 1)
    points_p = jnp.concatenate([yy * zz, xx * zz, zz], axis=2)
    intr_inv = jnp.linalg.inv(intrinsics)
    points_c = jnp.matmul(intr_inv, jnp.transpose(points_p, (0, 2, 1)))
    homog = jnp.ones((b, 1, h * w), dtype=jnp.float32)
    points_c = jnp.concatenate([points_c, homog], axis=1)
    points_w = jnp.matmul(extrinsics[:3], points_c)
    return jnp.transpose(points_w, (0, 2, 1))[:, :, :3]


def kernel(depth, extrinsics, intrinsics, tsdf_volume, feature_volume,
           origin, resolution, gpu, weights_volume):
    b, h, w = depth.shape
    coords = _camera_coords(depth, extrinsics, intrinsics)
    coords_t = coords[0].T.reshape(3, h, w)
    eye = extrinsics[0, :3, 3]
    vec = jnp.concatenate([
        eye.reshape(-1),
        origin.reshape(-1).astype(jnp.float32),
        jnp.full((1,), resolution, jnp.float32),
    ])
    params = jnp.pad(vec, (0, 32 - vec.shape[0])).reshape(1, 32)
    idx, wt = _geom_call(params, coords_t)
    out2 = _sc_call()(
        idx.reshape(8, NPTS),
        wt.reshape(8, NPTS),
        tsdf_volume.reshape(-1),
        weights_volume.reshape(-1),
    )
    out = out2.reshape(2, N_POINTS, h * w).transpose(0, 2, 1)
    return out.reshape(2, 1, h * w, N_POINTS)


# speedup vs baseline: 42.7049x; 20.7430x over previous
import functools

import jax
import jax.numpy as jnp
from jax import lax
from jax.experimental import pallas as pl
from jax.experimental.pallas import tpu as pltpu
from jax.experimental.pallas import tpu_sc as plsc

N_POINTS = 9
N_HALF = (N_POINTS - 1) // 2

H = 480
W = 640
RB = 16
NPIX = H * W
NPTS = NPIX * N_POINTS
VS = 256
NW = 32
PER_TILE = NPTS // NW
CHUNK = 1920
NCHUNKS = PER_TILE // CHUNK


def _geom_body(pp_ref, co_ref, idx_ref, wt_ref):
    P = lambda j: pp_ref[0, j]
    i = pl.program_id(0)
    cox = co_ref[0]
    coy = co_ref[1]
    coz = co_ref[2]
    flat = (jax.lax.broadcasted_iota(jnp.int32, (RB, W), 0) + i * RB) * W \
        + jax.lax.broadcasted_iota(jnp.int32, (RB, W), 1)
    o0, o1, o2, res = P(3), P(4), P(5), P(6)
    cenx = (cox - o0) / res
    ceny = (coy - o1) / res
    cenz = (coz - o2) / res
    eyex = (P(0) - o0) / res
    eyey = (P(1) - o1) / res
    eyez = (P(2) - o2) / res
    dx = cenx - eyex
    dy = ceny - eyey
    dz = cenz - eyez
    nrm = jnp.sqrt(dx * dx + dy * dy + dz * dz)
    inv = 1.0 / jnp.maximum(nrm, 1e-12)
    dx = dx * inv
    dy = dy * inv
    dz = dz * inv

    def corner_data(p, size, scale):
        f = jnp.floor(p)
        a = p - f
        nb = jnp.where(a > 0, 1.0, 0.0)
        g0 = f
        g1 = f + nb
        out = []
        for g, wgt in ((g0, 1.0 - a), (g1, a)):
            valid = (g >= 0) & (g < size)
            gi = jnp.minimum(jnp.maximum(g, 0.0), size - 1).astype(jnp.int32)
            gi = jnp.clip(gi, 0, size - 1) * scale
            out.append((wgt, valid, gi))
        return out

    for n in range(N_POINTS):
        off = float(n - N_HALF)
        px = cenx + off * dx
        py = ceny + off * dy
        pz = cenz + off * dz
        X = corner_data(px, VS, VS * VS)
        Y = corner_data(py, VS, VS)
        Z = corner_data(pz, VS, 1)
        cidx = 0
        for ci in (0, 1):
            wx, vx, ix = X[ci]
            for cj in (0, 1):
                wy, vy, iy = Y[cj]
                wxy = wx * wy
                vxy = vx & vy
                ixy = ix + iy
                pid = flat + n * NPIX
                for ck in (0, 1):
                    wz, vz, iz = Z[ck]
                    vall = vxy & vz
                    wt = jnp.where(vall, wxy * wz, 0.0)
                    idx_ref[cidx, n] = jnp.where(vall, ixy + iz, pid)
                    wt_ref[cidx, n] = wt
                    cidx += 1


def _geom_call(params, coords_t):
    shape = (8, N_POINTS, H, W)
    return pl.pallas_call(
        _geom_body,
        grid=(H // RB,),
        in_specs=[
            pl.BlockSpec((1, 32), lambda i: (0, 0)),
            pl.BlockSpec((3, RB, W), lambda i: (0, i, 0)),
        ],
        out_specs=[
            pl.BlockSpec((8, N_POINTS, RB, W), lambda i: (0, 0, i, 0)),
            pl.BlockSpec((8, N_POINTS, RB, W), lambda i: (0, 0, i, 0)),
        ],
        out_shape=[
            jax.ShapeDtypeStruct(shape, jnp.int32),
            jax.ShapeDtypeStruct(shape, jnp.float32),
        ],
    )(params, coords_t)


def _sc_body(idx_hbm, wt_hbm, ts_hbm, wv_hbm, out_hbm, *refs):
    idx_v = refs[0:8]
    wt_v = refs[8:16]
    gv = refs[16:24]
    gw = refs[24:32]
    accv, accw, insem, gsem = refs[32:36]
    wid = lax.axis_index("s") * 2 + lax.axis_index("c")
    base = wid * PER_TILE

    def chunk_body(ch, _):
        off = pl.multiple_of(base + ch * CHUNK, 128)
        stage = []
        for k in range(8):
            stage.append(pltpu.async_copy(
                idx_hbm.at[k, pl.ds(off, CHUNK)], idx_v[k], insem))
            stage.append(pltpu.async_copy(
                wt_hbm.at[k, pl.ds(off, CHUNK)], wt_v[k], insem))
        for cp in stage:
            cp.wait()
        copies = []
        for k in range(8):
            copies.append(pltpu.async_copy(ts_hbm.at[idx_v[k]], gv[k], gsem))
            copies.append(pltpu.async_copy(wv_hbm.at[idx_v[k]], gw[k], gsem))
        for cp in copies:
            cp.wait()

        def inner(i, _):
            s = pl.ds(i * 16, 16)
            av = jnp.zeros((16,), jnp.float32)
            aw = jnp.zeros((16,), jnp.float32)
            for k in range(8):
                wtk = wt_v[k][s]
                av = av + wtk * gv[k][s]
                aw = aw + wtk * gw[k][s]
            accv[s] = av
            accw[s] = aw
            return 0

        lax.fori_loop(0, CHUNK // 16, inner, 0)
        pltpu.sync_copy(accv, out_hbm.at[0, pl.ds(off, CHUNK)])
        pltpu.sync_copy(accw, out_hbm.at[1, pl.ds(off, CHUNK)])
        return 0

    lax.fori_loop(0, NCHUNKS, chunk_body, 0)


@functools.cache
def _sc_call():
    scratch = (
        [pltpu.VMEM((CHUNK,), jnp.int32) for _ in range(8)]
        + [pltpu.VMEM((CHUNK,), jnp.float32) for _ in range(24)]
        + [pltpu.VMEM((CHUNK,), jnp.float32) for _ in range(2)]
        + [pltpu.SemaphoreType.DMA, pltpu.SemaphoreType.DMA]
    )
    return functools.partial(
        pl.kernel,
        mesh=plsc.VectorSubcoreMesh(core_axis_name="c", subcore_axis_name="s"),
        out_type=jax.ShapeDtypeStruct((2, NPTS), jnp.float32),
        scratch_types=scratch,
    )(_sc_body)


def _camera_coords(depth, extrinsics, intrinsics):
    b, h, w = depth.shape
    xx, yy = jnp.meshgrid(jnp.arange(h, dtype=jnp.float32),
                          jnp.arange(w, dtype=jnp.float32), indexing='ij')
    xx = jnp.broadcast_to(xx.reshape(1, h * w, 1), (b, h * w, 1))
    yy = jnp.broadcast_to(yy.reshape(1, h * w, 1), (b, h * w, 1))
    zz = depth.reshape(b, h * w, 1)
    points_p = jnp.concatenate([yy * zz, xx * zz, zz], axis=2)
    intr_inv = jnp.linalg.inv(intrinsics)
    points_c = jnp.matmul(intr_inv, jnp.transpose(points_p, (0, 2, 1)))
    homog = jnp.ones((b, 1, h * w), dtype=jnp.float32)
    points_c = jnp.concatenate([points_c, homog], axis=1)
    points_w = jnp.matmul(extrinsics[:3], points_c)
    return jnp.transpose(points_w, (0, 2, 1))[:, :, :3]


def kernel(depth, extrinsics, intrinsics, tsdf_volume, feature_volume,
           origin, resolution, gpu, weights_volume):
    b, h, w = depth.shape
    coords = _camera_coords(depth, extrinsics, intrinsics)
    coords_t = coords[0].T.reshape(3, h, w)
    eye = extrinsics[0, :3, 3]
    vec = jnp.concatenate([
        eye.reshape(-1),
        origin.reshape(-1).astype(jnp.float32),
        jnp.full((1,), resolution, jnp.float32),
    ])
    params = jnp.pad(vec, (0, 32 - vec.shape[0])).reshape(1, 32)
    idx, wt = _geom_call(params, coords_t)
    out2 = _sc_call()(
        idx.reshape(8, NPTS),
        wt.reshape(8, NPTS),
        tsdf_volume.reshape(-1),
        weights_volume.reshape(-1),
    )
    out = out2.reshape(2, N_POINTS, h * w).transpose(0, 2, 1)
    return out.reshape(2, 1, h * w, N_POINTS)


# speedup vs baseline: 43.3252x; 1.0145x over previous
import functools

import jax
import jax.numpy as jnp
from jax import lax
from jax.experimental import pallas as pl
from jax.experimental.pallas import tpu as pltpu
from jax.experimental.pallas import tpu_sc as plsc

N_POINTS = 9
N_HALF = (N_POINTS - 1) // 2

H = 480
W = 640
RB = 16
NPIX = H * W
NPTS = NPIX * N_POINTS
VS = 256
NW = 32
PER_TILE = NPTS // NW
CHUNK = 3456
NCHUNKS = PER_TILE // CHUNK


def _geom_body(pp_ref, co_ref, idx_ref, wt_ref):
    P = lambda j: pp_ref[0, j]
    i = pl.program_id(0)
    cox = co_ref[0]
    coy = co_ref[1]
    coz = co_ref[2]
    flat = (jax.lax.broadcasted_iota(jnp.int32, (RB, W), 0) + i * RB) * W \
        + jax.lax.broadcasted_iota(jnp.int32, (RB, W), 1)
    o0, o1, o2, res = P(3), P(4), P(5), P(6)
    cenx = (cox - o0) / res
    ceny = (coy - o1) / res
    cenz = (coz - o2) / res
    eyex = (P(0) - o0) / res
    eyey = (P(1) - o1) / res
    eyez = (P(2) - o2) / res
    dx = cenx - eyex
    dy = ceny - eyey
    dz = cenz - eyez
    nrm = jnp.sqrt(dx * dx + dy * dy + dz * dz)
    inv = 1.0 / jnp.maximum(nrm, 1e-12)
    dx = dx * inv
    dy = dy * inv
    dz = dz * inv

    def corner_data(p, size, scale):
        f = jnp.floor(p)
        a = p - f
        nb = jnp.where(a > 0, 1.0, 0.0)
        g0 = f
        g1 = f + nb
        out = []
        for g, wgt in ((g0, 1.0 - a), (g1, a)):
            valid = (g >= 0) & (g < size)
            gi = jnp.minimum(jnp.maximum(g, 0.0), size - 1).astype(jnp.int32)
            gi = jnp.clip(gi, 0, size - 1) * scale
            out.append((wgt, valid, gi))
        return out

    for n in range(N_POINTS):
        off = float(n - N_HALF)
        px = cenx + off * dx
        py = ceny + off * dy
        pz = cenz + off * dz
        X = corner_data(px, VS, VS * VS)
        Y = corner_data(py, VS, VS)
        Z = corner_data(pz, VS, 1)
        cidx = 0
        for ci in (0, 1):
            wx, vx, ix = X[ci]
            for cj in (0, 1):
                wy, vy, iy = Y[cj]
                wxy = wx * wy
                vxy = vx & vy
                ixy = ix + iy
                pid = flat + n * NPIX
                for ck in (0, 1):
                    wz, vz, iz = Z[ck]
                    vall = vxy & vz
                    wt = jnp.where(vall, wxy * wz, 0.0)
                    idx_ref[cidx, n] = jnp.where(vall, ixy + iz, pid)
                    wt_ref[cidx, n] = wt
                    cidx += 1


def _geom_call(params, coords_t):
    shape = (8, N_POINTS, H, W)
    return pl.pallas_call(
        _geom_body,
        grid=(H // RB,),
        in_specs=[
            pl.BlockSpec((1, 32), lambda i: (0, 0)),
            pl.BlockSpec((3, RB, W), lambda i: (0, i, 0)),
        ],
        out_specs=[
            pl.BlockSpec((8, N_POINTS, RB, W), lambda i: (0, 0, i, 0)),
            pl.BlockSpec((8, N_POINTS, RB, W), lambda i: (0, 0, i, 0)),
        ],
        out_shape=[
            jax.ShapeDtypeStruct(shape, jnp.int32),
            jax.ShapeDtypeStruct(shape, jnp.float32),
        ],
    )(params, coords_t)


def _sc_body(idx_hbm, wt_hbm, ts_hbm, wv_hbm, out_hbm, *refs):
    idx_v = refs[0:8]
    wt_v = refs[8:16]
    gv = refs[16:24]
    gw = refs[24:32]
    accv, accw, insem, gsem = refs[32:36]
    wid = lax.axis_index("s") * 2 + lax.axis_index("c")
    base = wid * PER_TILE

    def chunk_body(ch, _):
        off = pl.multiple_of(base + ch * CHUNK, 128)
        stage = []
        for k in range(8):
            stage.append(pltpu.async_copy(
                idx_hbm.at[k, pl.ds(off, CHUNK)], idx_v[k], insem))
            stage.append(pltpu.async_copy(
                wt_hbm.at[k, pl.ds(off, CHUNK)], wt_v[k], insem))
        for cp in stage:
            cp.wait()
        copies = []
        for k in range(8):
            copies.append(pltpu.async_copy(ts_hbm.at[idx_v[k]], gv[k], gsem))
            copies.append(pltpu.async_copy(wv_hbm.at[idx_v[k]], gw[k], gsem))
        for cp in copies:
            cp.wait()

        def inner(i, _):
            s = pl.ds(i * 16, 16)
            av = jnp.zeros((16,), jnp.float32)
            aw = jnp.zeros((16,), jnp.float32)
            for k in range(8):
                wtk = wt_v[k][s]
                av = av + wtk * gv[k][s]
                aw = aw + wtk * gw[k][s]
            accv[s] = av
            accw[s] = aw
            return 0

        lax.fori_loop(0, CHUNK // 16, inner, 0)
        pltpu.sync_copy(accv, out_hbm.at[0, pl.ds(off, CHUNK)])
        pltpu.sync_copy(accw, out_hbm.at[1, pl.ds(off, CHUNK)])
        return 0

    lax.fori_loop(0, NCHUNKS, chunk_body, 0)


@functools.cache
def _sc_call():
    scratch = (
        [pltpu.VMEM((CHUNK,), jnp.int32) for _ in range(8)]
        + [pltpu.VMEM((CHUNK,), jnp.float32) for _ in range(24)]
        + [pltpu.VMEM((CHUNK,), jnp.float32) for _ in range(2)]
        + [pltpu.SemaphoreType.DMA, pltpu.SemaphoreType.DMA]
    )
    return functools.partial(
        pl.kernel,
        mesh=plsc.VectorSubcoreMesh(core_axis_name="c", subcore_axis_name="s"),
        out_type=jax.ShapeDtypeStruct((2, NPTS), jnp.float32),
        scratch_types=scratch,
    )(_sc_body)


def _camera_coords(depth, extrinsics, intrinsics):
    b, h, w = depth.shape
    xx, yy = jnp.meshgrid(jnp.arange(h, dtype=jnp.float32),
                          jnp.arange(w, dtype=jnp.float32), indexing='ij')
    xx = jnp.broadcast_to(xx.reshape(1, h * w, 1), (b, h * w, 1))
    yy = jnp.broadcast_to(yy.reshape(1, h * w, 1), (b, h * w, 1))
    zz = depth.reshape(b, h * w, 1)
    points_p = jnp.concatenate([yy * zz, xx * zz, zz], axis=2)
    intr_inv = jnp.linalg.inv(intrinsics)
    points_c = jnp.matmul(intr_inv, jnp.transpose(points_p, (0, 2, 1)))
    homog = jnp.ones((b, 1, h * w), dtype=jnp.float32)
    points_c = jnp.concatenate([points_c, homog], axis=1)
    points_w = jnp.matmul(extrinsics[:3], points_c)
    return jnp.transpose(points_w, (0, 2, 1))[:, :, :3]


def kernel(depth, extrinsics, intrinsics, tsdf_volume, feature_volume,
           origin, resolution, gpu, weights_volume):
    b, h, w = depth.shape
    coords = _camera_coords(depth, extrinsics, intrinsics)
    coords_t = coords[0].T.reshape(3, h, w)
    eye = extrinsics[0, :3, 3]
    vec = jnp.concatenate([
        eye.reshape(-1),
        origin.reshape(-1).astype(jnp.float32),
        jnp.full((1,), resolution, jnp.float32),
    ])
    params = jnp.pad(vec, (0, 32 - vec.shape[0])).reshape(1, 32)
    idx, wt = _geom_call(params, coords_t)
    out2 = _sc_call()(
        idx.reshape(8, NPTS),
        wt.reshape(8, NPTS),
        tsdf_volume.reshape(-1),
        weights_volume.reshape(-1),
    )
    out = out2.reshape(2, N_POINTS, h * w).transpose(0, 2, 1)
    return out.reshape(2, 1, h * w, N_POINTS)
